# baseline (speedup 1.0000x reference)
import functools

import jax
import jax.numpy as jnp
from jax import lax
from jax.experimental import pallas as pl
from jax.experimental.pallas import tpu as pltpu
from jax.experimental.pallas import tpu_sc as plsc

_N = 10000
_E = 320000
_NC = 2
_NS = 16
_CH = 128
_EPAD = 327680
_SUP = 20
_AGG_ROWS = _N + 128
_BN_EPS = 1e-5


def _staged_rows(s, rows0, copy_chunk):
  @pl.when(s < _NS - 1)
  def _():
    def body(k, carry):
      copy_chunk(s * 640 + k * _CH, _CH, rows0)
      return carry

    lax.fori_loop(0, 5, body, 0)

  @pl.when(s == _NS - 1)
  def _():
    def body(k, carry):
      copy_chunk(9600 + k * _CH, _CH, rows0)
      return carry

    lax.fori_loop(0, 3, body, 0)
    copy_chunk(9984, 16, rows0)


def _edge_loop(tbl_hbm, agg, idx_hbm, c, s, idx0, idx1, rows0, rows1,
               sem0, sem1, semi0, semi1, nchunk):
  nsuper = nchunk // _SUP
  blk = 2 * _SUP

  def idx_start(sb, buf, sem):
    pltpu.async_copy(idx_hbm.at[c, s, pl.ds(sb * blk, blk)], buf, sem)

  def idx_wait(buf, sem):
    pltpu.make_async_copy(idx_hbm.at[c, s, pl.ds(0, blk)], buf, sem).wait()

  def run_super(idxb):
    pltpu.async_copy(tbl_hbm.at[idxb.at[0]], rows0, sem0)

    def pair(k, carry):
      j = 2 * k
      pltpu.async_copy(tbl_hbm.at[idxb.at[2 * j + 2]], rows1, sem1)
      pltpu.make_async_copy(tbl_hbm.at[idxb.at[0]], rows0, sem0).wait()
      pltpu.sync_copy(rows0, agg.at[idxb.at[2 * j + 1]], add=True)

      @pl.when(j + 2 < _SUP)
      def _():
        pltpu.async_copy(tbl_hbm.at[idxb.at[2 * j + 4]], rows0, sem0)

      pltpu.make_async_copy(tbl_hbm.at[idxb.at[0]], rows1, sem1).wait()
      pltpu.sync_copy(rows1, agg.at[idxb.at[2 * j + 3]], add=True)
      return carry

    lax.fori_loop(0, _SUP // 2, pair, 0)

  idx_start(0, idx0, semi0)

  def super_pair(i, carry):
    sb = 2 * i
    idx_wait(idx0, semi0)
    idx_start(sb + 1, idx1, semi1)
    run_super(idx0)
    idx_wait(idx1, semi1)

    @pl.when(sb + 2 < nsuper)
    def _():
      idx_start(sb + 2, idx0, semi0)

    run_super(idx1)
    return carry

  lax.fori_loop(0, nsuper // 2, super_pair, 0)


def _sc_scratch(nblk):
  return dict(
      agg=pltpu.VMEM_SHARED((_AGG_ROWS, 128), jnp.float32),
      idx0=pltpu.VMEM((nblk, _CH), jnp.int32),
      idx1=pltpu.VMEM((nblk, _CH), jnp.int32),
      rows0=pltpu.VMEM((_CH, 128), jnp.float32),
      rows1=pltpu.VMEM((_CH, 128), jnp.float32),
      sem0=pltpu.SemaphoreType.DMA,
      sem1=pltpu.SemaphoreType.DMA,
      semi0=pltpu.SemaphoreType.DMA,
      semi1=pltpu.SemaphoreType.DMA,
  )


def _sc_aggregate_colsplit(tbl2d, idx):
  nchunk = _EPAD // (_NS * _CH)
  mesh = plsc.VectorSubcoreMesh(core_axis_name="c", subcore_axis_name="s")

  @functools.partial(
      pl.kernel,
      out_type=jax.ShapeDtypeStruct((_NC, _N, 128), jnp.float32),
      mesh=mesh,
      scratch_types=_sc_scratch(2 * _SUP),
  )
  def body(tbl2d_hbm, idx_hbm, out_hbm,
           agg, idx0, idx1, rows0, rows1, sem0, sem1, semi0, semi1):
    c = lax.axis_index("c")
    s = lax.axis_index("s")

    def init_chunk(off, sz, buf):
      base = pl.multiple_of(c * _N + off, 8)
      pltpu.sync_copy(tbl2d_hbm.at[pl.ds(base, sz)], buf.at[pl.ds(0, sz)])
      pltpu.sync_copy(buf.at[pl.ds(0, sz)], agg.at[pl.ds(off, sz)])

    _staged_rows(s, rows0, init_chunk)
    plsc.subcore_barrier()
    _edge_loop(tbl2d_hbm, agg, idx_hbm, c, s, idx0, idx1, rows0, rows1,
               sem0, sem1, semi0, semi1, nchunk)
    plsc.subcore_barrier()

    def out_chunk(off, sz, buf):
      pltpu.sync_copy(agg.at[pl.ds(off, sz)], buf.at[pl.ds(0, sz)])
      pltpu.sync_copy(buf.at[pl.ds(0, sz)], out_hbm.at[c, pl.ds(off, sz)])

    _staged_rows(s, rows0, out_chunk)

  return body(tbl2d, idx)


def _sc_aggregate_edgesplit(tbl, zer, idx):
  nchunk = _EPAD // (_NC * _NS * _CH)
  mesh = plsc.VectorSubcoreMesh(core_axis_name="c", subcore_axis_name="s")

  @functools.partial(
      pl.kernel,
      out_type=jax.ShapeDtypeStruct((_NC, _N, 128), jnp.float32),
      mesh=mesh,
      scratch_types=_sc_scratch(2 * _SUP),
  )
  def body(tbl_hbm, zer_hbm, idx_hbm, out_hbm,
           agg, idx0, idx1, rows0, rows1, sem0, sem1, semi0, semi1):
    c = lax.axis_index("c")
    s = lax.axis_index("s")

    @pl.when(c == 0)
    def _():
      def init_chunk(off, sz, buf):
        pltpu.sync_copy(tbl_hbm.at[pl.ds(off, sz)], buf.at[pl.ds(0, sz)])
        pltpu.sync_copy(buf.at[pl.ds(0, sz)], agg.at[pl.ds(off, sz)])

      _staged_rows(s, rows0, init_chunk)

    @pl.when(c == 1)
    def _():
      def init_chunk(off, sz, buf):
        pltpu.sync_copy(zer_hbm.at[pl.ds(off, sz)], buf.at[pl.ds(0, sz)])
        pltpu.sync_copy(buf.at[pl.ds(0, sz)], agg.at[pl.ds(off, sz)])

      _staged_rows(s, rows0, init_chunk)

    plsc.subcore_barrier()
    _edge_loop(tbl_hbm, agg, idx_hbm, c, s, idx0, idx1, rows0, rows1,
               sem0, sem1, semi0, semi1, nchunk)
    plsc.subcore_barrier()

    def out_chunk(off, sz, buf):
      pltpu.sync_copy(agg.at[pl.ds(off, sz)], buf.at[pl.ds(0, sz)])
      pltpu.sync_copy(buf.at[pl.ds(0, sz)], out_hbm.at[c, pl.ds(off, sz)])

    _staged_rows(s, rows0, out_chunk)

  return body(tbl, zer, idx)


def _mlp_finish(y, sc_ref, be_ref, out_ref, rs_ref):
  y = jnp.maximum(y, 0.0)
  y = y * sc_ref[...] + be_ref[...]
  out_ref[0] = y[:, :128]
  out_ref[1] = y[:, 128:]
  rs_ref[...] = jnp.sum(y, axis=1, keepdims=True)


def _mlp_sum_body(hpa_ref, w1_ref, b1_ref, w2_ref, b2_ref, sc_ref, be_ref,
                  out_ref, rs_ref):
  h = hpa_ref[0] + hpa_ref[1]
  y = jnp.dot(h, w1_ref[...], preferred_element_type=jnp.float32) + b1_ref[...]
  y = jnp.maximum(y, 0.0)
  y = jnp.dot(y, w2_ref[...], preferred_element_type=jnp.float32) + b2_ref[...]
  _mlp_finish(y, sc_ref, be_ref, out_ref, rs_ref)


def _mlp_halves_body(hpa_ref, w1_ref, b1_ref, w2_ref, b2_ref, sc_ref, be_ref,
                     out_ref, rs_ref):
  w1 = w1_ref[...]
  y = (jnp.dot(hpa_ref[0], w1[:128], preferred_element_type=jnp.float32)
       + jnp.dot(hpa_ref[1], w1[128:], preferred_element_type=jnp.float32)
       + b1_ref[...])
  y = jnp.maximum(y, 0.0)
  y = jnp.dot(y, w2_ref[...], preferred_element_type=jnp.float32) + b2_ref[...]
  _mlp_finish(y, sc_ref, be_ref, out_ref, rs_ref)


def _tc_mlp(hpa, w1, b1, w2, b2, scale, be, sum_planes):
  blk = 400
  grid = _N // blk
  d = w1.shape[0]
  hid = w2.shape[1]
  full = lambda i: (0, 0)
  body = _mlp_sum_body if sum_planes else _mlp_halves_body
  out, rs = pl.pallas_call(
      body,
      grid=(grid,),
      in_specs=[
          pl.BlockSpec((_NC, blk, 128), lambda i: (0, i, 0)),
          pl.BlockSpec((d, hid), full),
          pl.BlockSpec((1, hid), full),
          pl.BlockSpec((hid, hid), full),
          pl.BlockSpec((1, hid), full),
          pl.BlockSpec((1, hid), full),
          pl.BlockSpec((1, hid), full),
      ],
      out_specs=[
          pl.BlockSpec((_NC, blk, 128), lambda i: (0, i, 0)),
          pl.BlockSpec((blk, 1), lambda i: (i, 0)),
      ],
      out_shape=[
          jax.ShapeDtypeStruct((_NC, _N, 128), jnp.float32),
          jax.ShapeDtypeStruct((_N, 1), jnp.float32),
      ],
  )(hpa, w1, b1.reshape(1, hid), w2, b2.reshape(1, hid),
    scale.reshape(1, hid), be.reshape(1, hid))
  return out, rs[:, 0]


def kernel(x, edge_index, graph_len, W1_0, b1_0, W2_0, b2_0, g_0, be_0,
           W1_1, b1_1, W2_1, b2_1, g_1, be_1, W1_2, b1_2, W2_2, b2_2,
           g_2, be_2):
  del graph_len
  src = edge_index[0]
  dst = edge_index[1]
  npad = _EPAD - _E
  src_p = jnp.concatenate([src, jnp.zeros((npad,), jnp.int32)])
  pad_dst = _N + (jnp.arange(npad, dtype=jnp.int32) % 128)
  dst_p = jnp.concatenate([dst, pad_dst])
  nch_cs = _EPAD // (_NS * _CH)
  s_cs = src_p.reshape(_NS, nch_cs, _CH)
  d_cs = dst_p.reshape(_NS, nch_cs, _CH)
  idx_cs = jnp.stack([
      jnp.stack([s_cs, d_cs], axis=2).reshape(_NS, 2 * nch_cs, _CH),
      jnp.stack([s_cs + _N, d_cs], axis=2).reshape(_NS, 2 * nch_cs, _CH),
  ])
  nch_es = _EPAD // (_NC * _NS * _CH)
  s_es = src_p.reshape(_NC, _NS, nch_es, _CH)
  d_es = dst_p.reshape(_NC, _NS, nch_es, _CH)
  idx_es = jnp.stack([s_es, d_es], axis=3).reshape(_NC, _NS, 2 * nch_es, _CH)

  inv = 1.0 / jnp.sqrt(jnp.float32(1.0 + _BN_EPS))
  params = [
      (W1_0, b1_0, W2_0, b2_0, g_0 * inv, be_0),
      (W1_1, b1_1, W2_1, b2_1, g_1 * inv, be_1),
      (W1_2, b1_2, W2_2, b2_2, g_2 * inv, be_2),
  ]

  zer = jnp.zeros((_N, 128), jnp.float32)
  hs, pools = [], []
  h = None
  for li, (w1, b1, w2, b2, scale, be) in enumerate(params):
    if li == 0:
      hpa = _sc_aggregate_edgesplit(x, zer, idx_es)
      h, rs = _tc_mlp(hpa, w1, b1, w2, b2, scale, be, sum_planes=True)
    else:
      hpa = _sc_aggregate_colsplit(h.reshape(2 * _N, 128), idx_cs)
      h, rs = _tc_mlp(hpa, w1, b1, w2, b2, scale, be, sum_planes=False)
    hs.append(h)
    pools.append(rs)
  xcat = jnp.concatenate([hl[c] for hl in hs for c in range(_NC)], axis=-1)
  return jnp.concatenate(pools, axis=-1), xcat


# speedup vs baseline: 2.6138x; 2.6138x over previous
import functools

import jax
import jax.numpy as jnp
from jax import lax
from jax.experimental import pallas as pl
from jax.experimental.pallas import tpu as pltpu
from jax.experimental.pallas import tpu_sc as plsc

_N = 10000
_E = 320000
_NC = 2
_NS = 16
_CH = 128
_EPAD = 327680
_SUP = 20
_AGG_ROWS = _N + 128
_BN_EPS = 1e-5


def _staged_rows(s, rows0, copy_chunk):
  @pl.when(s < _NS - 1)
  def _():
    def body(k, carry):
      copy_chunk(s * 640 + k * _CH, _CH, rows0)
      return carry

    lax.fori_loop(0, 5, body, 0)

  @pl.when(s == _NS - 1)
  def _():
    def body(k, carry):
      copy_chunk(9600 + k * _CH, _CH, rows0)
      return carry

    lax.fori_loop(0, 3, body, 0)
    copy_chunk(9984, 16, rows0)


def _edge_loop(tbl_hbm, agg, idx_hbm, c, s, idx0, idx1, rows0, rows1,
               sem0, sem1, semi0, semi1, nchunk):
  nsuper = nchunk // _SUP
  blk = 2 * _SUP

  def idx_start(sb, buf, sem):
    pltpu.async_copy(idx_hbm.at[c, s, pl.ds(sb * blk, blk)], buf, sem)

  def idx_wait(buf, sem):
    pltpu.make_async_copy(idx_hbm.at[c, s, pl.ds(0, blk)], buf, sem).wait()

  def run_super(idxb):
    pltpu.async_copy(tbl_hbm.at[idxb.at[0]], rows0, sem0)

    def pair(k, carry):
      j = 2 * k
      pltpu.async_copy(tbl_hbm.at[idxb.at[2 * j + 2]], rows1, sem1)
      pltpu.make_async_copy(tbl_hbm.at[idxb.at[0]], rows0, sem0).wait()
      pltpu.sync_copy(rows0, agg.at[idxb.at[2 * j + 1]], add=True)

      @pl.when(j + 2 < _SUP)
      def _():
        pltpu.async_copy(tbl_hbm.at[idxb.at[2 * j + 4]], rows0, sem0)

      pltpu.make_async_copy(tbl_hbm.at[idxb.at[0]], rows1, sem1).wait()
      pltpu.sync_copy(rows1, agg.at[idxb.at[2 * j + 3]], add=True)
      return carry

    lax.fori_loop(0, _SUP // 2, pair, 0)

  idx_start(0, idx0, semi0)

  def super_pair(i, carry):
    sb = 2 * i
    idx_wait(idx0, semi0)
    idx_start(sb + 1, idx1, semi1)
    run_super(idx0)
    idx_wait(idx1, semi1)

    @pl.when(sb + 2 < nsuper)
    def _():
      idx_start(sb + 2, idx0, semi0)

    run_super(idx1)
    return carry

  lax.fori_loop(0, nsuper // 2, super_pair, 0)


def _sc_scratch(nblk):
  return dict(
      agg=pltpu.VMEM_SHARED((_AGG_ROWS, 128), jnp.float32),
      idx0=pltpu.VMEM((nblk, _CH), jnp.int32),
      idx1=pltpu.VMEM((nblk, _CH), jnp.int32),
      rows0=pltpu.VMEM((_CH, 128), jnp.float32),
      rows1=pltpu.VMEM((_CH, 128), jnp.float32),
      sem0=pltpu.SemaphoreType.DMA,
      sem1=pltpu.SemaphoreType.DMA,
      semi0=pltpu.SemaphoreType.DMA,
      semi1=pltpu.SemaphoreType.DMA,
  )


def _sc_aggregate_colsplit(tbl2d, idx):
  nchunk = _EPAD // (_NS * _CH)
  mesh = plsc.VectorSubcoreMesh(core_axis_name="c", subcore_axis_name="s")

  @functools.partial(
      pl.kernel,
      out_type=jax.ShapeDtypeStruct((_NC, _N, 128), jnp.float32),
      mesh=mesh,
      scratch_types=_sc_scratch(2 * _SUP),
  )
  def body(tbl2d_hbm, idx_hbm, out_hbm,
           agg, idx0, idx1, rows0, rows1, sem0, sem1, semi0, semi1):
    c = lax.axis_index("c")
    s = lax.axis_index("s")

    def init_chunk(off, sz, buf):
      base = pl.multiple_of(c * _N + off, 8)
      pltpu.sync_copy(tbl2d_hbm.at[pl.ds(base, sz)], buf.at[pl.ds(0, sz)])
      pltpu.sync_copy(buf.at[pl.ds(0, sz)], agg.at[pl.ds(off, sz)])

    _staged_rows(s, rows0, init_chunk)
    plsc.subcore_barrier()
    _edge_loop(tbl2d_hbm, agg, idx_hbm, c, s, idx0, idx1, rows0, rows1,
               sem0, sem1, semi0, semi1, nchunk)
    plsc.subcore_barrier()

    def out_chunk(off, sz, buf):
      pltpu.sync_copy(agg.at[pl.ds(off, sz)], buf.at[pl.ds(0, sz)])
      pltpu.sync_copy(buf.at[pl.ds(0, sz)], out_hbm.at[c, pl.ds(off, sz)])

    _staged_rows(s, rows0, out_chunk)

  return body(tbl2d, idx)


def _sc_aggregate_edgesplit(tbl, zer, idx):
  nchunk = _EPAD // (_NC * _NS * _CH)
  mesh = plsc.VectorSubcoreMesh(core_axis_name="c", subcore_axis_name="s")

  @functools.partial(
      pl.kernel,
      out_type=jax.ShapeDtypeStruct((_NC, _N, 128), jnp.float32),
      mesh=mesh,
      scratch_types=_sc_scratch(2 * _SUP),
  )
  def body(tbl_hbm, zer_hbm, idx_hbm, out_hbm,
           agg, idx0, idx1, rows0, rows1, sem0, sem1, semi0, semi1):
    c = lax.axis_index("c")
    s = lax.axis_index("s")

    @pl.when(c == 0)
    def _():
      def init_chunk(off, sz, buf):
        pltpu.sync_copy(tbl_hbm.at[pl.ds(off, sz)], buf.at[pl.ds(0, sz)])
        pltpu.sync_copy(buf.at[pl.ds(0, sz)], agg.at[pl.ds(off, sz)])

      _staged_rows(s, rows0, init_chunk)

    @pl.when(c == 1)
    def _():
      def init_chunk(off, sz, buf):
        pltpu.sync_copy(zer_hbm.at[pl.ds(off, sz)], buf.at[pl.ds(0, sz)])
        pltpu.sync_copy(buf.at[pl.ds(0, sz)], agg.at[pl.ds(off, sz)])

      _staged_rows(s, rows0, init_chunk)

    plsc.subcore_barrier()
    _edge_loop(tbl_hbm, agg, idx_hbm, c, s, idx0, idx1, rows0, rows1,
               sem0, sem1, semi0, semi1, nchunk)
    plsc.subcore_barrier()

    def out_chunk(off, sz, buf):
      pltpu.sync_copy(agg.at[pl.ds(off, sz)], buf.at[pl.ds(0, sz)])
      pltpu.sync_copy(buf.at[pl.ds(0, sz)], out_hbm.at[c, pl.ds(off, sz)])

    _staged_rows(s, rows0, out_chunk)

  return body(tbl, zer, idx)


def _mlp_finish(y, sc_ref, be_ref, out_ref, rs_ref):
  y = jnp.maximum(y, 0.0)
  y = y * sc_ref[...] + be_ref[...]
  out_ref[0] = y[:, :128]
  out_ref[1] = y[:, 128:]
  rs_ref[...] = jnp.sum(y, axis=1, keepdims=True)


def _mlp_sum_body(hpa_ref, w1_ref, b1_ref, w2_ref, b2_ref, sc_ref, be_ref,
                  out_ref, rs_ref):
  h = hpa_ref[0] + hpa_ref[1]
  y = jnp.dot(h, w1_ref[...], preferred_element_type=jnp.float32) + b1_ref[...]
  y = jnp.maximum(y, 0.0)
  y = jnp.dot(y, w2_ref[...], preferred_element_type=jnp.float32) + b2_ref[...]
  _mlp_finish(y, sc_ref, be_ref, out_ref, rs_ref)


def _mlp_halves_body(hpa_ref, w1_ref, b1_ref, w2_ref, b2_ref, sc_ref, be_ref,
                     out_ref, rs_ref):
  w1 = w1_ref[...]
  y = (jnp.dot(hpa_ref[0], w1[:128], preferred_element_type=jnp.float32)
       + jnp.dot(hpa_ref[1], w1[128:], preferred_element_type=jnp.float32)
       + b1_ref[...])
  y = jnp.maximum(y, 0.0)
  y = jnp.dot(y, w2_ref[...], preferred_element_type=jnp.float32) + b2_ref[...]
  _mlp_finish(y, sc_ref, be_ref, out_ref, rs_ref)


def _tc_mlp(hpa, w1, b1, w2, b2, scale, be, sum_planes):
  blk = 400
  grid = _N // blk
  d = w1.shape[0]
  hid = w2.shape[1]
  full = lambda i: (0, 0)
  body = _mlp_sum_body if sum_planes else _mlp_halves_body
  out, rs = pl.pallas_call(
      body,
      grid=(grid,),
      in_specs=[
          pl.BlockSpec((_NC, blk, 128), lambda i: (0, i, 0)),
          pl.BlockSpec((d, hid), full),
          pl.BlockSpec((1, hid), full),
          pl.BlockSpec((hid, hid), full),
          pl.BlockSpec((1, hid), full),
          pl.BlockSpec((1, hid), full),
          pl.BlockSpec((1, hid), full),
      ],
      out_specs=[
          pl.BlockSpec((_NC, blk, 128), lambda i: (0, i, 0)),
          pl.BlockSpec((blk, 1), lambda i: (i, 0)),
      ],
      out_shape=[
          jax.ShapeDtypeStruct((_NC, _N, 128), jnp.float32),
          jax.ShapeDtypeStruct((_N, 1), jnp.float32),
      ],
  )(hpa, w1, b1.reshape(1, hid), w2, b2.reshape(1, hid),
    scale.reshape(1, hid), be.reshape(1, hid))
  return out, rs[:, 0]


def kernel(x, edge_index, graph_len, W1_0, b1_0, W2_0, b2_0, g_0, be_0,
           W1_1, b1_1, W2_1, b2_1, g_1, be_1, W1_2, b1_2, W2_2, b2_2,
           g_2, be_2):
  del graph_len
  src = edge_index[0]
  dst = edge_index[1]
  npad = _EPAD - _E
  pad_iota = jnp.arange(npad, dtype=jnp.int32)
  src_p = jnp.concatenate([src, (pad_iota * 521) % _N])
  dst_p = jnp.concatenate([dst, _N + pad_iota % 128])
  nch_cs = _EPAD // (_NS * _CH)
  s_cs = src_p.reshape(_NS, nch_cs, _CH)
  d_cs = dst_p.reshape(_NS, nch_cs, _CH)
  idx_cs = jnp.stack([
      jnp.stack([s_cs, d_cs], axis=2).reshape(_NS, 2 * nch_cs, _CH),
      jnp.stack([s_cs + _N, d_cs], axis=2).reshape(_NS, 2 * nch_cs, _CH),
  ])
  nch_es = _EPAD // (_NC * _NS * _CH)
  s_es = src_p.reshape(_NC, _NS, nch_es, _CH)
  d_es = dst_p.reshape(_NC, _NS, nch_es, _CH)
  idx_es = jnp.stack([s_es, d_es], axis=3).reshape(_NC, _NS, 2 * nch_es, _CH)

  inv = 1.0 / jnp.sqrt(jnp.float32(1.0 + _BN_EPS))
  params = [
      (W1_0, b1_0, W2_0, b2_0, g_0 * inv, be_0),
      (W1_1, b1_1, W2_1, b2_1, g_1 * inv, be_1),
      (W1_2, b1_2, W2_2, b2_2, g_2 * inv, be_2),
  ]

  zer = jnp.zeros((_N, 128), jnp.float32)
  hs, pools = [], []
  h = None
  for li, (w1, b1, w2, b2, scale, be) in enumerate(params):
    if li == 0:
      hpa = _sc_aggregate_edgesplit(x, zer, idx_es)
      h, rs = _tc_mlp(hpa, w1, b1, w2, b2, scale, be, sum_planes=True)
    else:
      hpa = _sc_aggregate_colsplit(h.reshape(2 * _N, 128), idx_cs)
      h, rs = _tc_mlp(hpa, w1, b1, w2, b2, scale, be, sum_planes=False)
    hs.append(h)
    pools.append(rs)
  xcat = jnp.concatenate([hl[c] for hl in hs for c in range(_NC)], axis=-1)
  return jnp.concatenate(pools, axis=-1), xcat
